# baseline (speedup 1.0000x reference)
import jax
import jax.numpy as jnp
from jax.experimental import pallas as pl
from jax.experimental.pallas import tpu as pltpu

B, NCH, H, W = 16, 64, 32, 32
NPTS = H * W
BOOK = 1024
PTILE = 256
ROWS = PTILE // W


def _vq_body(c_ref, prec_ref, ze_ref, book_ref, zq_ref, prob_ref, logp_ref,
             bnorm_ref):
    prec = prec_ref[0]
    p_step = pl.program_id(1)
    book = book_ref[0]

    @pl.when(p_step == 0)
    def _():
        bnorm_ref[0, :] = prec * jnp.sum(book * book, axis=1)

    ze_t = ze_ref[0].reshape(NCH, PTILE)
    zep = ze_t.T * (2.0 * prec)
    g = (jnp.dot(zep, book.T, preferred_element_type=jnp.float32)
         - bnorm_ref[0, :][None, :])
    m = jnp.max(g, axis=1, keepdims=True)
    idx = jnp.argmax(g, axis=1)
    sh = g - m
    e = jnp.exp(sh)
    s = jnp.sum(e, axis=1, keepdims=True)
    prob_ref[0] = e * (1.0 / s)
    logp_ref[0] = sh - jnp.log(s)
    enc = (jax.lax.broadcasted_iota(jnp.int32, (PTILE, BOOK), 1)
           == idx[:, None]).astype(jnp.float32)
    zq = jnp.dot(enc, book, preferred_element_type=jnp.float32)
    zq_ref[0] = zq.T


def kernel(ze, c_logits, books, log_param_q, log_param_q_cls):
    param_q = 1.0 + jnp.exp(log_param_q)
    precision_q = 0.5 / jnp.clip(param_q, 1e-10)
    c = jnp.argmax(c_logits, axis=-1).astype(jnp.int32)
    prec_arr = jnp.reshape(precision_q, (1,)).astype(jnp.float32)

    grid = (B, NPTS // PTILE)
    grid_spec = pltpu.PrefetchScalarGridSpec(
        num_scalar_prefetch=2,
        grid=grid,
        in_specs=[
            pl.BlockSpec((1, NCH, ROWS, W), lambda b, p, c_r, q_r: (b, 0, p, 0)),
            pl.BlockSpec((1, BOOK, NCH), lambda b, p, c_r, q_r: (c_r[b], 0, 0)),
        ],
        out_specs=[
            pl.BlockSpec((1, NCH, PTILE), lambda b, p, c_r, q_r: (b, 0, p)),
            pl.BlockSpec((1, PTILE, BOOK), lambda b, p, c_r, q_r: (b, p, 0)),
            pl.BlockSpec((1, PTILE, BOOK), lambda b, p, c_r, q_r: (b, p, 0)),
        ],
        scratch_shapes=[pltpu.VMEM((1, BOOK), jnp.float32)],
    )
    zq_t, prob, log_prob = pl.pallas_call(
        _vq_body,
        grid_spec=grid_spec,
        out_shape=[
            jax.ShapeDtypeStruct((B, NCH, NPTS), jnp.float32),
            jax.ShapeDtypeStruct((B, NPTS, BOOK), jnp.float32),
            jax.ShapeDtypeStruct((B, NPTS, BOOK), jnp.float32),
        ],
    )(c, prec_arr, ze, books)
    zq = zq_t.reshape(B, NCH, H, W)
    return (zq, precision_q, prob, log_prob)


# speedup vs baseline: 1.1499x; 1.1499x over previous
import jax
import jax.numpy as jnp
from jax.experimental import pallas as pl
from jax.experimental.pallas import tpu as pltpu

B, NCH, H, W = 16, 64, 32, 32
NPTS = H * W
BOOK = 1024
PTILE = 512
ROWS = PTILE // W


def _vq_body(c_ref, prec_ref, ze_ref, book_ref, zq_ref, prob_ref, logp_ref,
             bnorm_ref):
    prec = prec_ref[0]
    p_step = pl.program_id(1)
    book = book_ref[0]

    @pl.when(p_step == 0)
    def _():
        bnorm_ref[0, :] = prec * jnp.sum(book * book, axis=1)

    ze_t = ze_ref[0].reshape(NCH, PTILE)
    zep = ze_t.T * (2.0 * prec)
    g = (jnp.dot(zep, book.T, preferred_element_type=jnp.float32)
         - bnorm_ref[0, :][None, :])
    m = jnp.max(g, axis=1, keepdims=True)
    idx = jnp.argmax(g, axis=1)
    sh = g - m
    e = jnp.exp(sh)
    s = jnp.sum(e, axis=1, keepdims=True)
    prob_ref[0] = e * (1.0 / s)
    logp_ref[0] = sh - jnp.log(s)
    enc = (jax.lax.broadcasted_iota(jnp.int32, (PTILE, BOOK), 1)
           == idx[:, None]).astype(jnp.float32)
    zq = jnp.dot(enc, book, preferred_element_type=jnp.float32)
    zq_ref[0] = zq.T


def kernel(ze, c_logits, books, log_param_q, log_param_q_cls):
    param_q = 1.0 + jnp.exp(log_param_q)
    precision_q = 0.5 / jnp.clip(param_q, 1e-10)
    c = jnp.argmax(c_logits, axis=-1).astype(jnp.int32)
    prec_arr = jnp.reshape(precision_q, (1,)).astype(jnp.float32)

    grid = (B, NPTS // PTILE)
    grid_spec = pltpu.PrefetchScalarGridSpec(
        num_scalar_prefetch=2,
        grid=grid,
        in_specs=[
            pl.BlockSpec((1, NCH, ROWS, W), lambda b, p, c_r, q_r: (b, 0, p, 0)),
            pl.BlockSpec((1, BOOK, NCH), lambda b, p, c_r, q_r: (c_r[b], 0, 0)),
        ],
        out_specs=[
            pl.BlockSpec((1, NCH, PTILE), lambda b, p, c_r, q_r: (b, 0, p)),
            pl.BlockSpec((1, PTILE, BOOK), lambda b, p, c_r, q_r: (b, p, 0)),
            pl.BlockSpec((1, PTILE, BOOK), lambda b, p, c_r, q_r: (b, p, 0)),
        ],
        scratch_shapes=[pltpu.VMEM((1, BOOK), jnp.float32)],
    )
    zq_t, prob, log_prob = pl.pallas_call(
        _vq_body,
        grid_spec=grid_spec,
        out_shape=[
            jax.ShapeDtypeStruct((B, NCH, NPTS), jnp.float32),
            jax.ShapeDtypeStruct((B, NPTS, BOOK), jnp.float32),
            jax.ShapeDtypeStruct((B, NPTS, BOOK), jnp.float32),
        ],
    )(c, prec_arr, ze, books)
    zq = zq_t.reshape(B, NCH, H, W)
    return (zq, precision_q, prob, log_prob)


# speedup vs baseline: 1.1575x; 1.0067x over previous
import jax
import jax.numpy as jnp
from jax.experimental import pallas as pl
from jax.experimental.pallas import tpu as pltpu

B, NCH, H, W = 16, 64, 32, 32
NPTS = H * W
BOOK = 1024
PTILE = 1024
ROWS = PTILE // W


def _vq_body(c_ref, prec_ref, ze_ref, book_ref, zq_ref, prob_ref, logp_ref,
             bnorm_ref):
    prec = prec_ref[0]
    p_step = pl.program_id(1)
    book = book_ref[0]

    @pl.when(p_step == 0)
    def _():
        bnorm_ref[0, :] = prec * jnp.sum(book * book, axis=1)

    ze_t = ze_ref[0].reshape(NCH, PTILE)
    zep = ze_t.T * (2.0 * prec)
    g = (jnp.dot(zep, book.T, preferred_element_type=jnp.float32)
         - bnorm_ref[0, :][None, :])
    m = jnp.max(g, axis=1, keepdims=True)
    idx = jnp.argmax(g, axis=1)
    sh = g - m
    e = jnp.exp(sh)
    s = jnp.sum(e, axis=1, keepdims=True)
    prob_ref[0] = e * (1.0 / s)
    logp_ref[0] = sh - jnp.log(s)
    enc = (jax.lax.broadcasted_iota(jnp.int32, (PTILE, BOOK), 1)
           == idx[:, None]).astype(jnp.float32)
    zq = jnp.dot(enc, book, preferred_element_type=jnp.float32)
    zq_ref[0] = zq.T


def kernel(ze, c_logits, books, log_param_q, log_param_q_cls):
    param_q = 1.0 + jnp.exp(log_param_q)
    precision_q = 0.5 / jnp.clip(param_q, 1e-10)
    c = jnp.argmax(c_logits, axis=-1).astype(jnp.int32)
    prec_arr = jnp.reshape(precision_q, (1,)).astype(jnp.float32)

    grid = (B, NPTS // PTILE)
    grid_spec = pltpu.PrefetchScalarGridSpec(
        num_scalar_prefetch=2,
        grid=grid,
        in_specs=[
            pl.BlockSpec((1, NCH, ROWS, W), lambda b, p, c_r, q_r: (b, 0, p, 0)),
            pl.BlockSpec((1, BOOK, NCH), lambda b, p, c_r, q_r: (c_r[b], 0, 0)),
        ],
        out_specs=[
            pl.BlockSpec((1, NCH, PTILE), lambda b, p, c_r, q_r: (b, 0, p)),
            pl.BlockSpec((1, PTILE, BOOK), lambda b, p, c_r, q_r: (b, p, 0)),
            pl.BlockSpec((1, PTILE, BOOK), lambda b, p, c_r, q_r: (b, p, 0)),
        ],
        scratch_shapes=[pltpu.VMEM((1, BOOK), jnp.float32)],
    )
    zq_t, prob, log_prob = pl.pallas_call(
        _vq_body,
        grid_spec=grid_spec,
        out_shape=[
            jax.ShapeDtypeStruct((B, NCH, NPTS), jnp.float32),
            jax.ShapeDtypeStruct((B, NPTS, BOOK), jnp.float32),
            jax.ShapeDtypeStruct((B, NPTS, BOOK), jnp.float32),
        ],
    )(c, prec_arr, ze, books)
    zq = zq_t.reshape(B, NCH, H, W)
    return (zq, precision_q, prob, log_prob)
